# Initial kernel scaffold; baseline (speedup 1.0000x reference)
import jax
import jax.numpy as jnp
from jax.experimental import pallas as pl


def kernel(x, prototypes, Wa0, Wa1, Wa2, Wa3, Wa4, Wa5, Wa6, Wb0, Wb1, Wb2, Wb3, Wb4, Wb5, Wb6):
    raise NotImplementedError("write your pallas kernel here")


# speedup vs baseline: 5.3010x; 5.3010x over previous
import functools

import jax
import jax.numpy as jnp
from jax.experimental import pallas as pl
from jax.experimental.pallas import tpu as pltpu

_DIM = 2048
_LORA_DIMS = (8, 16, 32, 48, 64, 96, 128)
_NE = 7
_STACK = 512
_BOUNDS = (8, 24, 56, 104, 168, 264, 392)
_TOKENS = 8192
_BLK = 512

_EXP_PREC = jax.lax.Precision.DEFAULT


def _moe_body(x_ref, p_ref, a_ref, b_ref, o_ref):
    x = x_ref[...]
    logits = jax.lax.dot_general(
        x.astype(jnp.bfloat16), p_ref[...].astype(jnp.bfloat16),
        (((1,), (1,)), ((), ())),
        preferred_element_type=jnp.float32,
        precision=jax.lax.Precision.DEFAULT)
    col8 = jax.lax.broadcasted_iota(jnp.int32, logits.shape, 1)
    logits = jnp.where(col8 < _NE, logits, jnp.float32(-3e38))
    m = jnp.max(logits, axis=1, keepdims=True)
    e = jnp.min(jnp.where(logits >= m, col8, _NE), axis=1, keepdims=True)

    h = jax.lax.dot_general(
        x, a_ref[...], (((1,), (0,)), ((), ())),
        preferred_element_type=jnp.float32, precision=_EXP_PREC)
    col = jax.lax.broadcasted_iota(jnp.int32, h.shape, 1)
    seg = jnp.zeros(h.shape, jnp.int32)
    for b in _BOUNDS:
        seg += (col >= b).astype(jnp.int32)
    hm = jnp.where(seg == e, h, jnp.float32(0.0))

    o_ref[...] = jax.lax.dot_general(
        hm, b_ref[...], (((1,), (0,)), ((), ())),
        preferred_element_type=jnp.float32, precision=_EXP_PREC)


@functools.partial(jax.jit, static_argnums=())
def kernel(x, prototypes, Wa0, Wa1, Wa2, Wa3, Wa4, Wa5, Wa6,
           Wb0, Wb1, Wb2, Wb3, Wb4, Wb5, Wb6):
    was = [Wa0, Wa1, Wa2, Wa3, Wa4, Wa5, Wa6]
    wbs = [Wb0, Wb1, Wb2, Wb3, Wb4, Wb5, Wb6]
    a_all = jnp.concatenate(was, axis=0)
    a_t = jnp.pad(a_all, ((0, _STACK - a_all.shape[0]), (0, 0))).T
    b_all = jnp.concatenate([w.T for w in wbs], axis=0)
    b_t = jnp.pad(b_all, ((0, _STACK - b_all.shape[0]), (0, 0)))
    p_pad = jnp.pad(prototypes, ((0, 8 - _NE), (0, 0)))

    n_blk = x.shape[0] // _BLK
    return pl.pallas_call(
        _moe_body,
        grid=(n_blk,),
        in_specs=[
            pl.BlockSpec((_BLK, _DIM), lambda i: (i, 0)),
            pl.BlockSpec((8, _DIM), lambda i: (0, 0)),
            pl.BlockSpec((_DIM, _STACK), lambda i: (0, 0)),
            pl.BlockSpec((_STACK, _DIM), lambda i: (0, 0)),
        ],
        out_specs=pl.BlockSpec((_BLK, _DIM), lambda i: (i, 0)),
        out_shape=jax.ShapeDtypeStruct((x.shape[0], _DIM), jnp.float32),
        compiler_params=pltpu.CompilerParams(
            dimension_semantics=("parallel",)),
    )(x, p_pad, a_t, b_t)


# speedup vs baseline: 5.4343x; 1.0251x over previous
import functools

import jax
import jax.numpy as jnp
from jax.experimental import pallas as pl
from jax.experimental.pallas import tpu as pltpu

_DIM = 2048
_LORA_DIMS = (8, 16, 32, 48, 64, 96, 128)
_NE = 7
_STACK = 512
_BOUNDS = (8, 24, 56, 104, 168, 264, 392)
_TOKENS = 8192
_BLK = 512

_EXP_PREC = jax.lax.Precision.DEFAULT


def _moe_body(x_ref, p_ref, a_ref, b_ref, o_ref):
    x = x_ref[...]
    logits = jax.lax.dot_general(
        x.astype(jnp.bfloat16), p_ref[...].astype(jnp.bfloat16),
        (((1,), (1,)), ((), ())),
        preferred_element_type=jnp.float32,
        precision=jax.lax.Precision.DEFAULT)
    col8 = jax.lax.broadcasted_iota(jnp.int32, logits.shape, 1)
    logits = jnp.where(col8 < _NE, logits, jnp.float32(-3e38))
    m = jnp.max(logits, axis=1, keepdims=True)
    e = jnp.min(jnp.where(logits >= m, col8, _NE), axis=1, keepdims=True)

    h = jax.lax.dot_general(
        x, a_ref[...], (((1,), (1,)), ((), ())),
        preferred_element_type=jnp.float32, precision=_EXP_PREC)
    col = jax.lax.broadcasted_iota(jnp.int32, h.shape, 1)
    seg = jnp.zeros(h.shape, jnp.int32)
    for b in _BOUNDS:
        seg += (col >= b).astype(jnp.int32)
    hm = jnp.where(seg == e, h, jnp.float32(0.0))

    o_ref[...] = jax.lax.dot_general(
        hm, b_ref[...], (((1,), (1,)), ((), ())),
        preferred_element_type=jnp.float32, precision=_EXP_PREC)


@functools.partial(jax.jit, static_argnums=())
def kernel(x, prototypes, Wa0, Wa1, Wa2, Wa3, Wa4, Wa5, Wa6,
           Wb0, Wb1, Wb2, Wb3, Wb4, Wb5, Wb6):
    was = [Wa0, Wa1, Wa2, Wa3, Wa4, Wa5, Wa6]
    wbs = [Wb0, Wb1, Wb2, Wb3, Wb4, Wb5, Wb6]
    a_all = jnp.concatenate(was, axis=0)
    a_pad = jnp.pad(a_all, ((0, _STACK - a_all.shape[0]), (0, 0)))
    b_all = jnp.concatenate(wbs, axis=1)
    b_pad = jnp.pad(b_all, ((0, 0), (0, _STACK - b_all.shape[1])))
    p_pad = jnp.pad(prototypes, ((0, 8 - _NE), (0, 0)))

    n_blk = x.shape[0] // _BLK
    return pl.pallas_call(
        _moe_body,
        grid=(n_blk,),
        in_specs=[
            pl.BlockSpec((_BLK, _DIM), lambda i: (i, 0)),
            pl.BlockSpec((8, _DIM), lambda i: (0, 0)),
            pl.BlockSpec((_STACK, _DIM), lambda i: (0, 0)),
            pl.BlockSpec((_DIM, _STACK), lambda i: (0, 0)),
        ],
        out_specs=pl.BlockSpec((_BLK, _DIM), lambda i: (i, 0)),
        out_shape=jax.ShapeDtypeStruct((x.shape[0], _DIM), jnp.float32),
        compiler_params=pltpu.CompilerParams(
            dimension_semantics=("parallel",)),
    )(x, p_pad, a_pad, b_pad)


# speedup vs baseline: 5.4928x; 1.0108x over previous
import functools

import jax
import jax.numpy as jnp
from jax.experimental import pallas as pl
from jax.experimental.pallas import tpu as pltpu

_DIM = 2048
_LORA_DIMS = (8, 16, 32, 48, 64, 96, 128)
_NE = 7
_STACK = 512
_BOUNDS = (8, 24, 56, 104, 168, 264, 392)
_TOKENS = 8192
_BLK = 1024

_EXP_PREC = jax.lax.Precision.DEFAULT


def _moe_body(x_ref, p_ref, a_ref, b_ref, o_ref):
    x = x_ref[...]
    logits = jax.lax.dot_general(
        x.astype(jnp.bfloat16), p_ref[...].astype(jnp.bfloat16),
        (((1,), (1,)), ((), ())),
        preferred_element_type=jnp.float32,
        precision=jax.lax.Precision.DEFAULT)
    col8 = jax.lax.broadcasted_iota(jnp.int32, logits.shape, 1)
    logits = jnp.where(col8 < _NE, logits, jnp.float32(-3e38))
    m = jnp.max(logits, axis=1, keepdims=True)
    e = jnp.min(jnp.where(logits >= m, col8, _NE), axis=1, keepdims=True)

    h = jax.lax.dot_general(
        x, a_ref[...], (((1,), (1,)), ((), ())),
        preferred_element_type=jnp.float32, precision=_EXP_PREC)
    col = jax.lax.broadcasted_iota(jnp.int32, h.shape, 1)
    seg = jnp.zeros(h.shape, jnp.int32)
    for b in _BOUNDS:
        seg += (col >= b).astype(jnp.int32)
    hm = jnp.where(seg == e, h, jnp.float32(0.0))

    o_ref[...] = jax.lax.dot_general(
        hm, b_ref[...], (((1,), (1,)), ((), ())),
        preferred_element_type=jnp.float32, precision=_EXP_PREC)


@functools.partial(jax.jit, static_argnums=())
def kernel(x, prototypes, Wa0, Wa1, Wa2, Wa3, Wa4, Wa5, Wa6,
           Wb0, Wb1, Wb2, Wb3, Wb4, Wb5, Wb6):
    was = [Wa0, Wa1, Wa2, Wa3, Wa4, Wa5, Wa6]
    wbs = [Wb0, Wb1, Wb2, Wb3, Wb4, Wb5, Wb6]
    a_all = jnp.concatenate(was, axis=0)
    a_pad = jnp.pad(a_all, ((0, _STACK - a_all.shape[0]), (0, 0)))
    b_all = jnp.concatenate(wbs, axis=1)
    b_pad = jnp.pad(b_all, ((0, 0), (0, _STACK - b_all.shape[1])))
    p_pad = jnp.pad(prototypes, ((0, 8 - _NE), (0, 0)))

    n_blk = x.shape[0] // _BLK
    return pl.pallas_call(
        _moe_body,
        grid=(n_blk,),
        in_specs=[
            pl.BlockSpec((_BLK, _DIM), lambda i: (i, 0)),
            pl.BlockSpec((8, _DIM), lambda i: (0, 0)),
            pl.BlockSpec((_STACK, _DIM), lambda i: (0, 0)),
            pl.BlockSpec((_DIM, _STACK), lambda i: (0, 0)),
        ],
        out_specs=pl.BlockSpec((_BLK, _DIM), lambda i: (i, 0)),
        out_shape=jax.ShapeDtypeStruct((x.shape[0], _DIM), jnp.float32),
        compiler_params=pltpu.CompilerParams(
            dimension_semantics=("parallel",)),
    )(x, p_pad, a_pad, b_pad)


# speedup vs baseline: 5.8145x; 1.0586x over previous
import jax
import jax.numpy as jnp
from jax.experimental import pallas as pl
from jax.experimental.pallas import tpu as pltpu

_DIM = 2048
_LORA_DIMS = (8, 16, 32, 48, 64, 96, 128)
_NE = 7
_STACK = 512
_STARTS = (0, 8, 24, 56, 104, 168, 264)
_BOUNDS = (8, 24, 56, 104, 168, 264, 392)
_BLK = 1024

_EXP_PREC = jax.lax.Precision.DEFAULT


def _moe_body(x_ref, p_ref, *rest):
    wa_refs = rest[0:_NE]
    wb_refs = rest[_NE:2 * _NE]
    o_ref = rest[2 * _NE]
    a_s = rest[2 * _NE + 1]
    b_s = rest[2 * _NE + 2]

    @pl.when(pl.program_id(0) == 0)
    def _assemble():
        a_s[...] = jnp.zeros_like(a_s)
        b_s[...] = jnp.zeros_like(b_s)
        for i in range(_NE):
            s, d = _STARTS[i], _LORA_DIMS[i]
            a_s[s:s + d, :] = wa_refs[i][...]
            b_s[:, s:s + d] = wb_refs[i][...]

    x = x_ref[...]
    logits = jax.lax.dot_general(
        x.astype(jnp.bfloat16), p_ref[...].astype(jnp.bfloat16),
        (((1,), (1,)), ((), ())),
        preferred_element_type=jnp.float32,
        precision=jax.lax.Precision.DEFAULT)
    col8 = jax.lax.broadcasted_iota(jnp.int32, logits.shape, 1)
    logits = jnp.where(col8 < _NE, logits, jnp.float32(-3e38))
    m = jnp.max(logits, axis=1, keepdims=True)
    e = jnp.min(jnp.where(logits >= m, col8, _NE), axis=1, keepdims=True)

    h = jax.lax.dot_general(
        x, a_s[...], (((1,), (1,)), ((), ())),
        preferred_element_type=jnp.float32, precision=_EXP_PREC)
    col = jax.lax.broadcasted_iota(jnp.int32, h.shape, 1)
    seg = jnp.zeros(h.shape, jnp.int32)
    for b in _BOUNDS:
        seg += (col >= b).astype(jnp.int32)
    hm = jnp.where(seg == e, h, jnp.float32(0.0))

    o_ref[...] = jax.lax.dot_general(
        hm, b_s[...], (((1,), (1,)), ((), ())),
        preferred_element_type=jnp.float32, precision=_EXP_PREC)


def kernel(x, prototypes, Wa0, Wa1, Wa2, Wa3, Wa4, Wa5, Wa6,
           Wb0, Wb1, Wb2, Wb3, Wb4, Wb5, Wb6):
    was = [Wa0, Wa1, Wa2, Wa3, Wa4, Wa5, Wa6]
    wbs = [Wb0, Wb1, Wb2, Wb3, Wb4, Wb5, Wb6]
    p_pad = jnp.pad(prototypes, ((0, 8 - _NE), (0, 0)))

    n_blk = x.shape[0] // _BLK
    const_spec = lambda shape: pl.BlockSpec(shape, lambda i: (0,) * len(shape))
    return pl.pallas_call(
        _moe_body,
        grid=(n_blk,),
        in_specs=[
            pl.BlockSpec((_BLK, _DIM), lambda i: (i, 0)),
            const_spec((8, _DIM)),
        ] + [const_spec((d, _DIM)) for d in _LORA_DIMS]
          + [const_spec((_DIM, d)) for d in _LORA_DIMS],
        out_specs=pl.BlockSpec((_BLK, _DIM), lambda i: (i, 0)),
        out_shape=jax.ShapeDtypeStruct((x.shape[0], _DIM), jnp.float32),
        scratch_shapes=[
            pltpu.VMEM((_STACK, _DIM), jnp.float32),
            pltpu.VMEM((_DIM, _STACK), jnp.float32),
        ],
        compiler_params=pltpu.CompilerParams(
            dimension_semantics=("arbitrary",)),
    )(x, p_pad, *was, *wbs)
